# baseline (speedup 1.0000x reference)
import functools

import jax
import jax.numpy as jnp
from jax import lax
from jax.experimental import pallas as pl
from jax.experimental.pallas import tpu as pltpu
from jax.experimental.pallas import tpu_sc as plsc

_N = 5000
_D = _H = _W = 192
_HW = _H * _W
_CHS = _D * _H * _W

_NC = 2
_NS = 16
_NW = _NC * _NS
_CHUNK = 256
_NPAD = _NW * _CHUNK
_G = _CHUNK // 16
_NIDX = 6 * _CHUNK
_ROWS = _NIDX // 128

_mesh = plsc.VectorSubcoreMesh(core_axis_name="c", subcore_axis_name="s")


@functools.partial(
    pl.kernel,
    mesh=_mesh,
    out_type=jax.ShapeDtypeStruct((3 * _NPAD,), jnp.float32),
    scratch_types=[
        pltpu.VMEM((3 * _CHUNK,), jnp.float32),
        pltpu.VMEM((3 * _CHUNK,), jnp.float32),
        pltpu.VMEM((128,), jnp.float32),
        pltpu.VMEM((_NIDX,), jnp.int32),
        pltpu.VMEM((_NIDX,), jnp.float32),
        pltpu.VMEM((3 * _CHUNK,), jnp.float32),
        pltpu.SemaphoreType.DMA,
    ],
)
def _trl_sc(fix_hbm, mov_hbm, field_hbm, sp_hbm, out_hbm,
            mv, fv, spv, idxv, valsv, ov, sem):
    wid = lax.axis_index("s") * _NC + lax.axis_index("c")
    base = wid * _CHUNK

    for ch in range(3):
        pltpu.sync_copy(mov_hbm.at[pl.ds(ch * _NPAD + base, _CHUNK)],
                        mv.at[pl.ds(ch * _CHUNK, _CHUNK)])
        pltpu.sync_copy(fix_hbm.at[pl.ds(ch * _NPAD + base, _CHUNK)],
                        fv.at[pl.ds(ch * _CHUNK, _CHUNK)])
    pltpu.sync_copy(sp_hbm, spv)

    for i in range(_G):
        x = mv[pl.ds(i * 16, 16)]
        y = mv[pl.ds(_CHUNK + i * 16, 16)]
        z = mv[pl.ds(2 * _CHUNK + i * 16, 16)]
        xf = x.astype(jnp.int32)
        yf = y.astype(jnp.int32)
        zf = z.astype(jnp.int32)
        xc = jnp.where(x > xf.astype(jnp.float32), xf + 1, xf)
        yc = jnp.where(y > yf.astype(jnp.float32), yf + 1, yf)
        zc = jnp.where(z > zf.astype(jnp.float32), zf + 1, zf)
        flat_f = xf * _HW + yf * _W + zf
        flat_c = xc * _HW + yc * _W + zc
        for ch in range(3):
            idxv[pl.ds(ch * _CHUNK + i * 16, 16)] = flat_f + ch * _CHS
            idxv[pl.ds((3 + ch) * _CHUNK + i * 16, 16)] = flat_c + ch * _CHS

    copies = [
        pltpu.async_copy(field_hbm.at[idxv.at[pl.ds(g * 128, 128)]],
                         valsv.at[pl.ds(g * 128, 128)], sem)
        for g in range(_ROWS)
    ]
    for cp in copies:
        cp.wait()

    for ch in range(3):
        sp = spv[pl.ds(ch * 16, 16)]
        for i in range(_G):
            o = ch * _CHUNK + i * 16
            f = valsv[pl.ds(o, 16)]
            c = valsv[pl.ds(3 * _CHUNK + o, 16)]
            disp = (f + c) * 0.5
            ov[pl.ds(o, 16)] = (mv[pl.ds(o, 16)] + disp - fv[pl.ds(o, 16)]) * sp
        pltpu.sync_copy(ov.at[pl.ds(ch * _CHUNK, _CHUNK)],
                        out_hbm.at[pl.ds(ch * _NPAD + base, _CHUNK)])


def kernel(fixed_landmarks, moving_landmarks, displacement_field,
           fixed_spacing, moving_spacing):
    mov_t = jnp.zeros((3, _NPAD), jnp.float32).at[:, :_N].set(
        moving_landmarks.T).reshape(3 * _NPAD)
    fix_t = jnp.zeros((3, _NPAD), jnp.float32).at[:, :_N].set(
        fixed_landmarks.T).reshape(3 * _NPAD)
    field_flat = displacement_field.reshape(3 * _CHS)
    sp_b = jnp.concatenate([
        jnp.broadcast_to(moving_spacing.reshape(3, 1), (3, 16)).reshape(48),
        jnp.zeros((80,), jnp.float32),
    ])
    out_pad = _trl_sc(fix_t, mov_t, field_flat, sp_b)
    return out_pad.reshape(3, _NPAD)[:, :_N].T
